# baseline (speedup 1.0000x reference)
import functools

import jax
import jax.numpy as jnp
from jax import lax
from jax.experimental import pallas as pl
from jax.experimental.pallas import tpu as pltpu
from jax.experimental.pallas import tpu_sc as plsc

N = 10000
C = 128
E = 320000
NC = 2
NS = 16
NW = NC * NS
EPW = E // NW
L = 16
CHUNK = 80
CPB = 25
BLK = CHUNK * CPB
NBLK = EPW // BLK
NPAD = 10240
RPT = NPAD // NS


def _tc_aux(h, att_s, att_d):
    asn = jnp.sum(h * att_s[None, :], axis=1)
    adn = jnp.sum(h * att_d[None, :], axis=1)
    shift = jnp.maximum(jnp.max(asn) + jnp.max(adn), 0.0)
    return asn, adn, jnp.full((L,), shift, jnp.float32)


def _tc_prep_body(x_ref, w_ref, as_ref, ad_ref,
                  h_ref, asn_ref, adn_ref, shift_ref):
    h = jnp.dot(x_ref[...], w_ref[...], preferred_element_type=jnp.float32)
    h_ref[...] = h
    asn_ref[...], adn_ref[...], shift_ref[...] = _tc_aux(
        h, as_ref[...], ad_ref[...])


def _tc_prep(x, W, att_s, att_d):
    return pl.pallas_call(
        _tc_prep_body,
        out_shape=(
            jax.ShapeDtypeStruct((N, C), jnp.float32),
            jax.ShapeDtypeStruct((N,), jnp.float32),
            jax.ShapeDtypeStruct((N,), jnp.float32),
            jax.ShapeDtypeStruct((L,), jnp.float32),
        ),
    )(x, W, att_s, att_d)


def _tc_mid_body(p_ref, d_ref, b_ref, w_ref, as_ref, ad_ref,
                 h_ref, asn_ref, adn_ref, shift_ref):
    num = (p_ref[0] + p_ref[1])[:N, :]
    den = jnp.sum(d_ref[...], axis=0)
    out1 = num / (den[:, None] + 1e-16) + b_ref[...][None, :]
    x2 = jnp.maximum(out1, 0.0)
    h = jnp.dot(x2, w_ref[...], preferred_element_type=jnp.float32)
    h_ref[...] = h
    asn_ref[...], adn_ref[...], shift_ref[...] = _tc_aux(
        h, as_ref[...], ad_ref[...])


def _tc_mid(parts, dens, b, W, att_s, att_d):
    return pl.pallas_call(
        _tc_mid_body,
        out_shape=(
            jax.ShapeDtypeStruct((N, C), jnp.float32),
            jax.ShapeDtypeStruct((N,), jnp.float32),
            jax.ShapeDtypeStruct((N,), jnp.float32),
            jax.ShapeDtypeStruct((L,), jnp.float32),
        ),
    )(parts, dens, b, W, att_s, att_d)


def _tc_final_body(p_ref, d_ref, b_ref, o_ref):
    num = (p_ref[0] + p_ref[1])[:N, :]
    den = jnp.sum(d_ref[...], axis=0)
    o_ref[...] = num / (den[:, None] + 1e-16) + b_ref[...][None, :]


def _tc_final(parts, dens, b):
    return pl.pallas_call(
        _tc_final_body,
        out_shape=jax.ShapeDtypeStruct((N, C), jnp.float32),
    )(parts, dens, b)


_mesh = plsc.VectorSubcoreMesh(core_axis_name="c", subcore_axis_name="s")


@functools.partial(
    pl.kernel,
    out_type=(
        jax.ShapeDtypeStruct((NC, NPAD, C), jnp.float32),
        jax.ShapeDtypeStruct((NW, N), jnp.float32),
    ),
    mesh=_mesh,
    compiler_params=pltpu.CompilerParams(needs_layout_passes=False),
    scratch_types=[
        pltpu.VMEM((CPB, CHUNK), jnp.int32),
        pltpu.VMEM((CPB, CHUNK), jnp.int32),
        pltpu.VMEM((EPW,), jnp.float32),
        pltpu.VMEM((L,), jnp.float32),
        pltpu.VMEM_SHARED((NPAD, C), jnp.float32),
        pltpu.SemaphoreType.DMA,
        pltpu.SemaphoreType.DMA,
        pltpu.SemaphoreType.DMA,
        pltpu.SemaphoreType.DMA,
        pltpu.SemaphoreType.DMA,
        pltpu.SemaphoreType.DMA,
    ],
)
def _sc_edge(h_hbm, asn_hbm, adn_hbm, shift_hbm, src_hbm, dst_hbm,
             msg_out, den_out,
             src_v, dst_v, w_v, shift_v, acc_sh,
             sg0, sg1, sg2, ss0, ss1, ss2):
    cid = lax.axis_index("c")
    sid = lax.axis_index("s")
    wid = cid * NS + sid
    base = wid * EPW
    r0 = sid * RPT
    sg = (sg0, sg1, sg2)
    ss = (ss0, ss1, ss2)

    pltpu.sync_copy(shift_hbm, shift_v)
    shift = shift_v[...][0]

    src2 = src_hbm
    dst2 = dst_hbm

    def _phase1(asn_v, adn_v, den_v):
        pltpu.sync_copy(asn_hbm, asn_v)
        pltpu.sync_copy(adn_hbm, adn_v)

        def _zden(g, c):
            den_v[pl.ds(g * L, L)] = jnp.zeros((L,), jnp.float32)
            return c
        lax.fori_loop(0, N // L, _zden, 0)

        def _blk1(bi, c):
            pltpu.sync_copy(src2.at[wid, bi], src_v)
            pltpu.sync_copy(dst2.at[wid, bi], dst_v)

            def _wrow(t, c2):
                def _wgrp(g, c3):
                    o = g * L
                    sidx = src_v[t, pl.ds(o, L)]
                    didx = dst_v[t, pl.ds(o, L)]
                    e = (plsc.load_gather(asn_v, [sidx])
                         + plsc.load_gather(adn_v, [didx]))
                    e = jnp.where(e < 0.0, e * jnp.float32(0.2), e)
                    w = jnp.exp(e - shift)
                    w_v[pl.ds(bi * BLK + t * CHUNK + o, L)] = w
                    plsc.addupdate_scatter(den_v, [didx], w)
                    return c3
                lax.fori_loop(0, CHUNK // L, _wgrp, 0)
                return c2
            lax.fori_loop(0, CPB, _wrow, 0)
            return c
        lax.fori_loop(0, NBLK, _blk1, 0)
        pltpu.sync_copy(den_v, den_out.at[wid])

    pl.run_scoped(_phase1,
                  pltpu.VMEM((N,), jnp.float32),
                  pltpu.VMEM((N,), jnp.float32),
                  pltpu.VMEM((N,), jnp.float32))

    def _phase2(rows0, rows1, rows2):
        rows = (rows0, rows1, rows2)

        def _zrow(r, c):
            for j in range(C // L):
                rows0[r, pl.ds(j * L, L)] = jnp.zeros((L,), jnp.float32)
            return c
        lax.fori_loop(0, CHUNK, _zrow, 0)
        for j in range(RPT // CHUNK):
            pltpu.sync_copy(rows0, acc_sh.at[pl.ds(r0 + j * CHUNK, CHUNK)])
        plsc.subcore_barrier()

        def _gather(t, b):
            pltpu.async_copy(h_hbm.at[src_v.at[t]], rows[b], sg[b])

        def _wait_gather(t, b):
            pltpu.make_async_copy(h_hbm.at[src_v.at[t]], rows[b], sg[b]).wait()

        def _scat(t, b):
            pltpu.async_copy(rows[b], acc_sh.at[dst_v.at[t]], ss[b], add=True)

        def _wait_scat(t, b):
            pltpu.make_async_copy(rows[b], acc_sh.at[dst_v.at[t]], ss[b]).wait()

        def _scale(bi, t, b):
            woff = bi * BLK + t * CHUNK

            def _grp(g, c2):
                roff = g * L
                w16 = w_v[pl.ds(woff + roff, L)]
                for rj in range(L):
                    r = roff + rj
                    wv = jnp.full((L,), w16[rj], jnp.float32)
                    for j in range(C // L):
                        rows[b][r, pl.ds(j * L, L)] = (
                            rows[b][r, pl.ds(j * L, L)] * wv)
                return c2
            lax.fori_loop(0, CHUNK // L, _grp, 0)

        def _blk2(bi, c):
            pltpu.sync_copy(src2.at[wid, bi], src_v)
            pltpu.sync_copy(dst2.at[wid, bi], dst_v)

            _gather(0, 0)
            _wait_gather(0, 0)
            _gather(1, 1)
            _scale(bi, 0, 0)
            _scat(0, 0)
            _wait_gather(1, 1)
            _gather(2, 2)
            _scale(bi, 1, 1)
            _scat(1, 1)

            def _trip(i, c2):
                for bb in range(3):
                    t = 2 + 3 * i + bb
                    b = (2 + bb) % 3
                    _wait_gather(t, b)
                    _wait_scat(t - 2, (b + 1) % 3)
                    _gather(t + 1, (b + 1) % 3)
                    _scale(bi, t, b)
                    _scat(t, b)
                return c2
            lax.fori_loop(0, 7, _trip, 0)

            _wait_gather(23, 2)
            _wait_scat(21, 0)
            _gather(24, 0)
            _scale(bi, 23, 2)
            _scat(23, 2)
            _wait_gather(24, 0)
            _wait_scat(22, 1)
            _scale(bi, 24, 0)
            _scat(24, 0)
            _wait_scat(23, 2)
            _wait_scat(24, 0)
            return c
        lax.fori_loop(0, NBLK, _blk2, 0)

    pl.run_scoped(_phase2,
                  pltpu.VMEM((CHUNK, C), jnp.float32),
                  pltpu.VMEM((CHUNK, C), jnp.float32),
                  pltpu.VMEM((CHUNK, C), jnp.float32))

    plsc.subcore_barrier()
    pltpu.sync_copy(acc_sh.at[pl.ds(r0, RPT)], msg_out.at[cid, pl.ds(r0, RPT)])


def kernel(x, edge_index, W1, att_src1, att_dst1, b1, W2, att_src2, att_dst2, b2):
    ei = edge_index.astype(jnp.int32)
    src = ei[0].reshape(NW, NBLK, CPB, CHUNK)
    dst = ei[1].reshape(NW, NBLK, CPB, CHUNK)
    h1, asn1, adn1, shift1 = _tc_prep(x, W1, att_src1, att_dst1)
    parts1, dens1 = _sc_edge(h1, asn1, adn1, shift1, src, dst)
    h2, asn2, adn2, shift2 = _tc_mid(parts1, dens1, b1, W2, att_src2, att_dst2)
    parts2, dens2 = _sc_edge(h2, asn2, adn2, shift2, src, dst)
    return _tc_final(parts2, dens2, b2)


# speedup vs baseline: 1.1678x; 1.1678x over previous
import functools

import jax
import jax.numpy as jnp
from jax import lax
from jax.experimental import pallas as pl
from jax.experimental.pallas import tpu as pltpu
from jax.experimental.pallas import tpu_sc as plsc

N = 10000
C = 128
E = 320000
NC = 2
NS = 16
NW = NC * NS
EPW = E // NW
L = 16
CHUNK = 80
CPB = 25
BLK = CHUNK * CPB
NBLK = EPW // BLK
NPAD = 10240
RPT = NPAD // NS


def _tc_aux(h, att_s, att_d):
    asn = jnp.sum(h * att_s[None, :], axis=1)
    adn = jnp.sum(h * att_d[None, :], axis=1)
    shift = jnp.maximum(jnp.max(asn) + jnp.max(adn), 0.0)
    return asn, adn, jnp.full((L,), shift, jnp.float32)


def _tc_prep_body(x_ref, w_ref, as_ref, ad_ref,
                  h_ref, asn_ref, adn_ref, shift_ref):
    h = jnp.dot(x_ref[...], w_ref[...], preferred_element_type=jnp.float32)
    h_ref[...] = h
    asn_ref[...], adn_ref[...], shift_ref[...] = _tc_aux(
        h, as_ref[...], ad_ref[...])


def _tc_prep(x, W, att_s, att_d):
    return pl.pallas_call(
        _tc_prep_body,
        out_shape=(
            jax.ShapeDtypeStruct((N, C), jnp.float32),
            jax.ShapeDtypeStruct((N,), jnp.float32),
            jax.ShapeDtypeStruct((N,), jnp.float32),
            jax.ShapeDtypeStruct((L,), jnp.float32),
        ),
    )(x, W, att_s, att_d)


def _tc_mid_body(p_ref, d_ref, b_ref, w_ref, as_ref, ad_ref,
                 h_ref, asn_ref, adn_ref, shift_ref):
    num = (p_ref[0] + p_ref[1])[:N, :]
    den = jnp.sum(d_ref[...], axis=0)
    out1 = num / (den[:, None] + 1e-16) + b_ref[...][None, :]
    x2 = jnp.maximum(out1, 0.0)
    h = jnp.dot(x2, w_ref[...], preferred_element_type=jnp.float32)
    h_ref[...] = h
    asn_ref[...], adn_ref[...], shift_ref[...] = _tc_aux(
        h, as_ref[...], ad_ref[...])


def _tc_mid(parts, dens, b, W, att_s, att_d):
    return pl.pallas_call(
        _tc_mid_body,
        out_shape=(
            jax.ShapeDtypeStruct((N, C), jnp.float32),
            jax.ShapeDtypeStruct((N,), jnp.float32),
            jax.ShapeDtypeStruct((N,), jnp.float32),
            jax.ShapeDtypeStruct((L,), jnp.float32),
        ),
    )(parts, dens, b, W, att_s, att_d)


def _tc_final_body(p_ref, d_ref, b_ref, o_ref):
    num = (p_ref[0] + p_ref[1])[:N, :]
    den = jnp.sum(d_ref[...], axis=0)
    o_ref[...] = num / (den[:, None] + 1e-16) + b_ref[...][None, :]


def _tc_final(parts, dens, b):
    return pl.pallas_call(
        _tc_final_body,
        out_shape=jax.ShapeDtypeStruct((N, C), jnp.float32),
    )(parts, dens, b)


_mesh = plsc.VectorSubcoreMesh(core_axis_name="c", subcore_axis_name="s")


@functools.partial(
    pl.kernel,
    out_type=(
        jax.ShapeDtypeStruct((NC, NPAD, C), jnp.float32),
        jax.ShapeDtypeStruct((NW, N), jnp.float32),
    ),
    mesh=_mesh,
    compiler_params=pltpu.CompilerParams(needs_layout_passes=False),
    scratch_types=[
        pltpu.VMEM((CPB, CHUNK), jnp.int32),
        pltpu.VMEM((CPB, CHUNK), jnp.int32),
        pltpu.VMEM((EPW,), jnp.float32),
        pltpu.VMEM((L,), jnp.float32),
        pltpu.VMEM_SHARED((NPAD, C), jnp.float32),
        pltpu.SemaphoreType.DMA,
        pltpu.SemaphoreType.DMA,
        pltpu.SemaphoreType.DMA,
        pltpu.SemaphoreType.DMA,
        pltpu.SemaphoreType.DMA,
        pltpu.SemaphoreType.DMA,
    ],
)
def _sc_edge(h_hbm, asn_hbm, adn_hbm, shift_hbm, src_hbm, dst_hbm,
             msg_out, den_out,
             src_v, dst_v, w_v, shift_v, acc_sh,
             sg0, sg1, sg2, ss0, ss1, ss2):
    cid = lax.axis_index("c")
    sid = lax.axis_index("s")
    wid = cid * NS + sid
    base = wid * EPW
    r0 = sid * RPT
    sg = (sg0, sg1, sg2)
    ss = (ss0, ss1, ss2)

    pltpu.sync_copy(shift_hbm, shift_v)
    shift = shift_v[...][0]

    src2 = src_hbm
    dst2 = dst_hbm

    def _phase1(asn_v, adn_v, den_v):
        pltpu.sync_copy(asn_hbm, asn_v)
        pltpu.sync_copy(adn_hbm, adn_v)

        def _zden(g, c):
            den_v[pl.ds(g * L, L)] = jnp.zeros((L,), jnp.float32)
            return c
        lax.fori_loop(0, N // L, _zden, 0)

        def _blk1(bi, c):
            pltpu.sync_copy(src2.at[wid, bi], src_v)
            pltpu.sync_copy(dst2.at[wid, bi], dst_v)

            def _wrow(t, c2):
                def _wgrp(g, c3):
                    o = g * L
                    sidx = src_v[t, pl.ds(o, L)]
                    didx = dst_v[t, pl.ds(o, L)]
                    e = (plsc.load_gather(asn_v, [sidx])
                         + plsc.load_gather(adn_v, [didx]))
                    e = jnp.where(e < 0.0, e * jnp.float32(0.2), e)
                    w = jnp.exp(e - shift)
                    w_v[pl.ds(bi * BLK + t * CHUNK + o, L)] = w
                    plsc.addupdate_scatter(den_v, [didx], w)
                    return c3
                lax.fori_loop(0, CHUNK // L, _wgrp, 0)
                return c2
            lax.fori_loop(0, CPB, _wrow, 0)
            return c
        lax.fori_loop(0, NBLK, _blk1, 0)
        pltpu.sync_copy(den_v, den_out.at[wid])

    pl.run_scoped(_phase1,
                  pltpu.VMEM((N,), jnp.float32),
                  pltpu.VMEM((N,), jnp.float32),
                  pltpu.VMEM((N,), jnp.float32))

    def _phase2(rows0, rows1, rows2):
        rows = (rows0, rows1, rows2)

        def _zrow(r, c):
            for j in range(C // L):
                rows0[r, pl.ds(j * L, L)] = jnp.zeros((L,), jnp.float32)
            return c
        lax.fori_loop(0, CHUNK, _zrow, 0)
        for j in range(RPT // CHUNK):
            pltpu.sync_copy(rows0, acc_sh.at[pl.ds(r0 + j * CHUNK, CHUNK)])
        plsc.subcore_barrier()

        def _gather(t, b):
            pltpu.async_copy(h_hbm.at[src_v.at[t]], rows[b], sg[b])

        def _wait_gather(t, b):
            pltpu.make_async_copy(h_hbm.at[src_v.at[t]], rows[b], sg[b]).wait()

        def _scat(t, b):
            pltpu.async_copy(rows[b], acc_sh.at[dst_v.at[t]], ss[b], add=True)

        def _wait_scat(t, b):
            pltpu.make_async_copy(rows[b], acc_sh.at[dst_v.at[t]], ss[b]).wait()

        def _scale(bi, t, b):
            woff = bi * BLK + t * CHUNK

            def _grp(g, c2):
                roff = g * L
                w16 = w_v[pl.ds(woff + roff, L)]
                for rj in range(L):
                    r = roff + rj
                    wv = jnp.full((L,), w16[rj], jnp.float32)
                    for j in range(C // L):
                        rows[b][r, pl.ds(j * L, L)] = (
                            rows[b][r, pl.ds(j * L, L)] * wv)
                return c2
            lax.fori_loop(0, CHUNK // L, _grp, 0)

        def _blk2(bi, c):
            pltpu.sync_copy(src2.at[wid, bi], src_v)
            pltpu.sync_copy(dst2.at[wid, bi], dst_v)

            _gather(0, 0)
            _gather(1, 1)
            _wait_gather(0, 0)
            _scale(bi, 0, 0)
            _scat(0, 0)
            _gather(2, 2)
            _wait_gather(1, 1)
            _scale(bi, 1, 1)
            _scat(1, 1)
            _wait_scat(0, 0)
            _gather(3, 0)

            def _trip(i, c2):
                for bb in range(3):
                    t = 2 + 3 * i + bb
                    b = (2 + bb) % 3
                    _wait_gather(t, b)
                    _scale(bi, t, b)
                    _scat(t, b)
                    _wait_scat(t - 1, (b + 2) % 3)
                    _gather(t + 2, (b + 2) % 3)
                return c2
            lax.fori_loop(0, 7, _trip, 0)

            _wait_gather(23, 2)
            _scale(bi, 23, 2)
            _scat(23, 2)
            _wait_scat(22, 1)
            _wait_gather(24, 0)
            _scale(bi, 24, 0)
            _scat(24, 0)
            _wait_scat(23, 2)
            _wait_scat(24, 0)
            return c
        lax.fori_loop(0, NBLK, _blk2, 0)

    pl.run_scoped(_phase2,
                  pltpu.VMEM((CHUNK, C), jnp.float32),
                  pltpu.VMEM((CHUNK, C), jnp.float32),
                  pltpu.VMEM((CHUNK, C), jnp.float32))

    plsc.subcore_barrier()
    pltpu.sync_copy(acc_sh.at[pl.ds(r0, RPT)], msg_out.at[cid, pl.ds(r0, RPT)])


def kernel(x, edge_index, W1, att_src1, att_dst1, b1, W2, att_src2, att_dst2, b2):
    ei = edge_index.astype(jnp.int32)
    src = ei[0].reshape(NW, NBLK, CPB, CHUNK)
    dst = ei[1].reshape(NW, NBLK, CPB, CHUNK)
    h1, asn1, adn1, shift1 = _tc_prep(x, W1, att_src1, att_dst1)
    parts1, dens1 = _sc_edge(h1, asn1, adn1, shift1, src, dst)
    h2, asn2, adn2, shift2 = _tc_mid(parts1, dens1, b1, W2, att_src2, att_dst2)
    parts2, dens2 = _sc_edge(h2, asn2, adn2, shift2, src, dst)
    return _tc_final(parts2, dens2, b2)
